# baseline (speedup 1.0000x reference)
import functools

import jax
import jax.numpy as jnp
from jax import lax
from jax.experimental import pallas as pl
from jax.experimental.pallas import tpu as pltpu
from jax.experimental.pallas import tpu_sc as plsc

F_SP = 26
F_DN = 13
NF = F_SP + F_DN
D = 16
V = 100000
VPAD = 100352
EPS = 1e-5

XF = 16

NC = 2
NS = 16
NW = NC * NS
CHUNK = 128
NG = CHUNK // 16

HALF_V = VPAD // 2
VW = VPAD // NW
NVG = VW // 16

_SC_PARAMS = pltpu.CompilerParams(
    use_tc_tiling_on_sc=False, needs_layout_passes=False
)
_MESH = dict(core_axis_name="c", subcore_axis_name="s")


def _sc_extract(inputs_flat):
    b = inputs_flat.shape[0] // NF
    ngrp = b // CHUNK
    nchunk = ngrp // NW

    @functools.partial(
        pl.kernel,
        out_type=(
            jax.ShapeDtypeStruct((ngrp, F_SP, CHUNK), jnp.int32),
            jax.ShapeDtypeStruct((ngrp, F_SP, CHUNK), jnp.int32),
        ),
        mesh=plsc.VectorSubcoreMesh(**_MESH),
        scratch_types=[
            pltpu.VMEM((CHUNK * NF,), jnp.float32),
            pltpu.VMEM((1, F_SP, CHUNK), jnp.int32),
            pltpu.VMEM((1, F_SP, CHUNK), jnp.int32),
        ],
        compiler_params=_SC_PARAMS,
    )
    def k(in_hbm, idx_hbm, off_hbm, raw_v, idx_v, off_v):
        wid = lax.axis_index("c") * NS + lax.axis_index("s")
        lane = lax.iota(jnp.int32, 16)

        @pl.loop(0, nchunk)
        def _chunk(kc):
            grp = wid * nchunk + kc
            pltpu.sync_copy(in_hbm.at[pl.ds(grp * CHUNK * NF, CHUNK * NF)], raw_v)

            for f in range(F_SP):
                base = (f if f < XF else f - XF) * VPAD
                for g in range(NG):
                    pos = lane * NF + (g * 16 * NF) + f
                    ids = plsc.load_gather(raw_v, [pos]).astype(jnp.int32)
                    p = ids + base
                    idx_v[0, f, pl.ds(g * 16, 16)] = p >> 4
                    off_v[0, f, pl.ds(g * 16, 16)] = p & 15

            pltpu.sync_copy(idx_v, idx_hbm.at[pl.ds(grp, 1)])
            pltpu.sync_copy(off_v, off_hbm.at[pl.ds(grp, 1)])

    return k(inputs_flat)


def _tc_rowsum(tables_t):
    def body(in_ref, out_ref):
        out_ref[...] = jnp.sum(in_ref[0], axis=0)

    return pl.pallas_call(
        body,
        grid=(XF, 2),
        in_specs=[pl.BlockSpec((1, D, HALF_V), lambda f, h: (f, 0, h))],
        out_specs=pl.BlockSpec((HALF_V,), lambda f, h: (f * 2 + h,)),
        out_shape=jax.ShapeDtypeStruct((XF * VPAD,), jnp.float32),
    )(tables_t)


def _sc_rowsum(tables_t):
    nfb = F_SP - XF

    @functools.partial(
        pl.kernel,
        out_type=jax.ShapeDtypeStruct((nfb * VPAD,), jnp.float32),
        mesh=plsc.VectorSubcoreMesh(**_MESH),
        scratch_types=[
            pltpu.VMEM((2, D, VW), jnp.float32),
            pltpu.VMEM((VW,), jnp.float32),
            pltpu.SemaphoreType.DMA,
        ],
        compiler_params=_SC_PARAMS,
    )
    def k(tab_hbm, out_hbm, buf_v, red_v, sem):
        wid = lax.axis_index("c") * NS + lax.axis_index("s")
        v0 = wid * VW

        def fetch(fb, slot):
            pltpu.async_copy(
                tab_hbm.at[pl.ds(XF + fb, 1), :, pl.ds(v0, VW)],
                buf_v.at[pl.ds(slot, 1)],
                sem,
            )

        def drain(slot):
            pltpu.make_async_copy(
                tab_hbm.at[pl.ds(XF, 1), :, pl.ds(v0, VW)],
                buf_v.at[pl.ds(slot, 1)],
                sem,
            ).wait()

        fetch(0, 0)

        @pl.loop(0, nfb)
        def _field(fb):
            slot = lax.rem(fb, 2)
            drain(slot)

            @pl.when(fb + 1 < nfb)
            def _():
                fetch(fb + 1, 1 - slot)

            @pl.loop(0, NVG)
            def _vg(vg):
                sl = pl.ds(vg * 16, 16)
                acc = buf_v[slot, 0, sl]
                a1 = buf_v[slot, 1, sl]
                a2 = buf_v[slot, 2, sl]
                a3 = buf_v[slot, 3, sl]
                for d in range(4, D, 4):
                    acc = acc + buf_v[slot, d, sl]
                    a1 = a1 + buf_v[slot, d + 1, sl]
                    a2 = a2 + buf_v[slot, d + 2, sl]
                    a3 = a3 + buf_v[slot, d + 3, sl]
                red_v[sl] = (acc + a1) + (a2 + a3)

            pltpu.sync_copy(red_v, out_hbm.at[pl.ds(fb * VPAD + v0, VW)])

    return k(tables_t)


def _sc_gather_sum(idx_all, off_all, s16a, s16b):
    ngrp = idx_all.shape[0]
    nchunk = ngrp // NW
    nrow = F_SP * CHUNK

    @functools.partial(
        pl.kernel,
        out_type=jax.ShapeDtypeStruct((ngrp, CHUNK), jnp.float32),
        mesh=plsc.VectorSubcoreMesh(**_MESH),
        scratch_types=[
            pltpu.VMEM((2, F_SP, CHUNK), jnp.int32),
            pltpu.VMEM((2, F_SP, CHUNK), jnp.int32),
            pltpu.VMEM((2 * nrow, D), jnp.float32),
            pltpu.VMEM((1, CHUNK), jnp.float32),
            pltpu.SemaphoreType.DMA,
            pltpu.SemaphoreType.DMA,
        ],
        compiler_params=_SC_PARAMS,
    )
    def k(idx_hbm, off_hbm, sa_hbm, sb_hbm, out_hbm,
          idx_v, off_v, rows_v, red_v, gsem, isem):
        wid = lax.axis_index("c") * NS + lax.axis_index("s")
        lane = lax.iota(jnp.int32, 16)
        grp0 = wid * nchunk

        def fetch_idx(kc, slot):
            pltpu.async_copy(
                idx_hbm.at[pl.ds(grp0 + kc, 1)], idx_v.at[pl.ds(slot, 1)], isem
            )
            pltpu.async_copy(
                off_hbm.at[pl.ds(grp0 + kc, 1)], off_v.at[pl.ds(slot, 1)], isem
            )

        def wait_idx():
            pltpu.make_async_copy(
                idx_hbm.at[pl.ds(0, 2)], idx_v, isem
            ).wait()

        def fire_gathers(kc, slot):
            @pl.loop(0, XF)
            def _fa(f):
                pltpu.async_copy(
                    sa_hbm.at[idx_v.at[slot, f]],
                    rows_v.at[pl.ds(slot * nrow + f * CHUNK, CHUNK), :],
                    gsem,
                )

            @pl.loop(XF, F_SP)
            def _fb(f):
                pltpu.async_copy(
                    sb_hbm.at[idx_v.at[slot, f]],
                    rows_v.at[pl.ds(slot * nrow + f * CHUNK, CHUNK), :],
                    gsem,
                )

        def drain_gathers():
            pltpu.make_async_copy(
                sa_hbm.at[pl.ds(0, nrow), :],
                rows_v.at[pl.ds(0, nrow), :],
                gsem,
            ).wait()

        fetch_idx(0, 0)
        wait_idx()
        fire_gathers(0, 0)
        if nchunk > 1:
            fetch_idx(1, 1)

        @pl.loop(0, nchunk)
        def _chunk(kc):
            slot = lax.rem(kc, 2)
            drain_gathers()

            @pl.when(kc + 1 < nchunk)
            def _():
                wait_idx()
                fire_gathers(kc + 1, 1 - slot)

            for g in range(NG):
                rbase = slot * nrow + lane + g * 16
                acc = plsc.load_gather(
                    rows_v, [rbase, off_v[slot, 0, pl.ds(g * 16, 16)]]
                )
                for f in range(1, F_SP):
                    acc = acc + plsc.load_gather(
                        rows_v,
                        [rbase + f * CHUNK, off_v[slot, f, pl.ds(g * 16, 16)]],
                    )
                red_v[0, pl.ds(g * 16, 16)] = acc

            @pl.when(kc + 2 < nchunk)
            def _():
                fetch_idx(kc + 2, slot)

            pltpu.sync_copy(red_v, out_hbm.at[pl.ds(grp0 + kc, 1), :])

    return k(idx_all, off_all, s16a, s16b)


def _tc_combine(inputs, sp, gamma, beta, wt, bias):
    def body(in_ref, sp_ref, g_ref, b_ref, w_ref, bias_ref, out_ref):
        d = in_ref[:, F_SP:]
        mean = jnp.mean(d, axis=0, keepdims=True)
        c = d - mean
        var = jnp.mean(c * c, axis=0, keepdims=True)
        bn = c * lax.rsqrt(var + EPS) * g_ref[...][None, :] + b_ref[...][None, :]
        dense_logit = jnp.sum(bn * w_ref[...], axis=1, keepdims=True)
        out_ref[...] = sp_ref[...] + dense_logit + bias_ref[...][None, :]

    return pl.pallas_call(
        body,
        out_shape=jax.ShapeDtypeStruct((inputs.shape[0], 1), jnp.float32),
    )(inputs, sp, gamma, beta, wt, bias)


def kernel(inputs, tables, gamma, beta, W, bias):
    b = inputs.shape[0]
    tables_t = jnp.transpose(tables, (0, 2, 1))
    idx_all, off_all = _sc_extract(inputs.reshape(-1))
    s_a = _tc_rowsum(tables_t)
    s_b = _sc_rowsum(tables_t)
    sp = _sc_gather_sum(
        idx_all,
        off_all,
        s_a.reshape(XF * VPAD // D, D),
        s_b.reshape((F_SP - XF) * VPAD // D, D),
    )
    wt = W.reshape(1, F_DN)
    return _tc_combine(inputs, sp.reshape(b, 1), gamma, beta, wt, bias)


# speedup vs baseline: 2.4351x; 2.4351x over previous
import functools

import jax
import jax.numpy as jnp
from jax import lax
from jax.experimental import pallas as pl
from jax.experimental.pallas import tpu as pltpu
from jax.experimental.pallas import tpu_sc as plsc

F_SP = 26
F_DN = 13
NF = F_SP + F_DN
D = 16
V = 100000
VPAD = 100352
EPS = 1e-5

NC = 2
NS = 16
NW = NC * NS
CHUNK = 128
NG = CHUNK // 16

HALF_V = VPAD // 2
FH = F_SP // 2
NROW16 = FH * VPAD // 16

_SC_PARAMS = pltpu.CompilerParams(
    use_tc_tiling_on_sc=False, needs_layout_passes=False
)
_MESH = dict(core_axis_name="c", subcore_axis_name="s")


def _sc_extract(inputs_flat):
    b = inputs_flat.shape[0] // NF
    ngrp = b // CHUNK
    nchunk = ngrp // NW

    @functools.partial(
        pl.kernel,
        out_type=(
            jax.ShapeDtypeStruct((ngrp, F_SP, CHUNK), jnp.int32),
            jax.ShapeDtypeStruct((ngrp, F_SP, CHUNK), jnp.int32),
        ),
        mesh=plsc.VectorSubcoreMesh(**_MESH),
        scratch_types=[
            pltpu.VMEM((CHUNK * NF,), jnp.float32),
            pltpu.VMEM((1, F_SP, CHUNK), jnp.int32),
            pltpu.VMEM((1, F_SP, CHUNK), jnp.int32),
        ],
        compiler_params=_SC_PARAMS,
    )
    def k(in_hbm, idx_hbm, off_hbm, raw_v, idx_v, off_v):
        wid = lax.axis_index("c") * NS + lax.axis_index("s")
        lane = lax.iota(jnp.int32, 16)

        @pl.loop(0, nchunk)
        def _chunk(kc):
            grp = wid * nchunk + kc
            pltpu.sync_copy(in_hbm.at[pl.ds(grp * CHUNK * NF, CHUNK * NF)], raw_v)

            @pl.loop(0, F_SP)
            def _field(f):
                for g in range(NG):
                    pos = lane * NF + (g * 16 * NF) + f
                    ids = plsc.load_gather(raw_v, [pos]).astype(jnp.int32)
                    p = ids + (f % FH) * VPAD
                    idx_v[0, f, pl.ds(g * 16, 16)] = p >> 4
                    off_v[0, f, pl.ds(g * 16, 16)] = p & 15

            pltpu.sync_copy(idx_v, idx_hbm.at[pl.ds(grp, 1)])
            pltpu.sync_copy(off_v, off_hbm.at[pl.ds(grp, 1)])

    return k(inputs_flat)


def _tc_rowsum(tables_t, f0):
    def body(in_ref, out_ref):
        out_ref[...] = jnp.sum(in_ref[0], axis=0)

    return pl.pallas_call(
        body,
        grid=(FH, 2),
        in_specs=[pl.BlockSpec((1, D, HALF_V), lambda f, h: (f0 + f, 0, h))],
        out_specs=pl.BlockSpec((HALF_V,), lambda f, h: (f * 2 + h,)),
        out_shape=jax.ShapeDtypeStruct((FH * VPAD,), jnp.float32),
    )(tables_t)


def _sc_gather_sum(idx_all, off_all, s16, f0):
    ngrp = idx_all.shape[0]
    nchunk = ngrp // NW
    nrow = FH * CHUNK

    @functools.partial(
        pl.kernel,
        out_type=jax.ShapeDtypeStruct((ngrp, CHUNK), jnp.float32),
        mesh=plsc.VectorSubcoreMesh(**_MESH),
        scratch_types=[
            pltpu.VMEM((2, FH, CHUNK), jnp.int32),
            pltpu.VMEM((2, FH, CHUNK), jnp.int32),
            pltpu.VMEM((2 * nrow, D), jnp.float32),
            pltpu.VMEM((1, CHUNK), jnp.float32),
            pltpu.SemaphoreType.DMA,
            pltpu.SemaphoreType.DMA,
        ],
        compiler_params=_SC_PARAMS,
    )
    def k(idx_hbm, off_hbm, s_hbm, out_hbm, idx_v, off_v, rows_v, red_v, gsem, isem):
        wid = lax.axis_index("c") * NS + lax.axis_index("s")
        lane = lax.iota(jnp.int32, 16)
        grp0 = wid * nchunk

        def fetch_idx(kc, slot):
            pltpu.async_copy(
                idx_hbm.at[pl.ds(grp0 + kc, 1), pl.ds(f0, FH), :],
                idx_v.at[pl.ds(slot, 1)],
                isem,
            )
            pltpu.async_copy(
                off_hbm.at[pl.ds(grp0 + kc, 1), pl.ds(f0, FH), :],
                off_v.at[pl.ds(slot, 1)],
                isem,
            )

        def wait_idx():
            pltpu.make_async_copy(
                idx_hbm.at[pl.ds(0, 2), pl.ds(f0, FH), :], idx_v, isem
            ).wait()

        def fire_gathers(kc, slot):
            @pl.loop(0, FH)
            def _field(f):
                pltpu.async_copy(
                    s_hbm.at[idx_v.at[slot, f]],
                    rows_v.at[pl.ds(slot * nrow + f * CHUNK, CHUNK), :],
                    gsem,
                )

        def drain_gathers():
            pltpu.make_async_copy(
                s_hbm.at[pl.ds(0, nrow), :],
                rows_v.at[pl.ds(0, nrow), :],
                gsem,
            ).wait()

        fetch_idx(0, 0)
        wait_idx()
        fire_gathers(0, 0)
        if nchunk > 1:
            fetch_idx(1, 1)

        @pl.loop(0, nchunk)
        def _chunk(kc):
            slot = lax.rem(kc, 2)
            drain_gathers()

            @pl.when(kc + 1 < nchunk)
            def _():
                wait_idx()
                fire_gathers(kc + 1, 1 - slot)

            for g in range(NG):
                rbase = slot * nrow + lane + g * 16
                acc = plsc.load_gather(
                    rows_v, [rbase, off_v[slot, 0, pl.ds(g * 16, 16)]]
                )
                for f in range(1, FH):
                    acc = acc + plsc.load_gather(
                        rows_v,
                        [rbase + f * CHUNK, off_v[slot, f, pl.ds(g * 16, 16)]],
                    )
                red_v[0, pl.ds(g * 16, 16)] = acc

            @pl.when(kc + 2 < nchunk)
            def _():
                fetch_idx(kc + 2, slot)

            pltpu.sync_copy(red_v, out_hbm.at[pl.ds(grp0 + kc, 1), :])

    return k(idx_all, off_all, s16)


def _tc_combine(inputs, sp_a, sp_b, gamma, beta, wt, bias):
    def body(in_ref, spa_ref, spb_ref, g_ref, b_ref, w_ref, bias_ref, out_ref):
        d = in_ref[:, F_SP:]
        mean = jnp.mean(d, axis=0, keepdims=True)
        c = d - mean
        var = jnp.mean(c * c, axis=0, keepdims=True)
        bn = c * lax.rsqrt(var + EPS) * g_ref[...][None, :] + b_ref[...][None, :]
        dense_logit = jnp.sum(bn * w_ref[...], axis=1, keepdims=True)
        out_ref[...] = (
            spa_ref[...] + spb_ref[...] + dense_logit + bias_ref[...][None, :]
        )

    return pl.pallas_call(
        body,
        out_shape=jax.ShapeDtypeStruct((inputs.shape[0], 1), jnp.float32),
    )(inputs, sp_a, sp_b, gamma, beta, wt, bias)


def kernel(inputs, tables, gamma, beta, W, bias):
    b = inputs.shape[0]
    tables_t = jnp.transpose(tables, (0, 2, 1))
    idx_all, off_all = _sc_extract(inputs.reshape(-1))
    s_a = _tc_rowsum(tables_t, 0)
    sp_a = _sc_gather_sum(idx_all, off_all, s_a.reshape(NROW16, D), 0)
    s_b = _tc_rowsum(tables_t, FH)
    sp_b = _sc_gather_sum(idx_all, off_all, s_b.reshape(NROW16, D), FH)
    wt = W.reshape(1, F_DN)
    return _tc_combine(
        inputs, sp_a.reshape(b, 1), sp_b.reshape(b, 1), gamma, beta, wt, bias
    )


# speedup vs baseline: 2.4496x; 1.0060x over previous
import functools

import jax
import jax.numpy as jnp
from jax import lax
from jax.experimental import pallas as pl
from jax.experimental.pallas import tpu as pltpu
from jax.experimental.pallas import tpu_sc as plsc

F_SP = 26
F_DN = 13
NF = F_SP + F_DN
D = 16
V = 100000
VPAD = 100352
EPS = 1e-5

NC = 2
NS = 16
NW = NC * NS
CHUNK = 128
NG = CHUNK // 16

HALF_V = VPAD // 2
NROW16 = F_SP * VPAD // 16

_SC_PARAMS = pltpu.CompilerParams(
    use_tc_tiling_on_sc=False, needs_layout_passes=False
)
_MESH = dict(core_axis_name="c", subcore_axis_name="s")


def _tc_rowsum(tables_t):
    def body(in_ref, out_ref):
        out_ref[...] = jnp.sum(in_ref[0], axis=0)

    return pl.pallas_call(
        body,
        grid=(F_SP, 2),
        in_specs=[pl.BlockSpec((1, D, HALF_V), lambda f, h: (f, 0, h))],
        out_specs=pl.BlockSpec((HALF_V,), lambda f, h: (f * 2 + h,)),
        out_shape=jax.ShapeDtypeStruct((F_SP * VPAD,), jnp.float32),
    )(tables_t)


def _sc_fused(inputs_flat, s16):
    b = inputs_flat.shape[0] // NF
    ngrp = b // CHUNK
    nchunk = ngrp // NW
    nrow = F_SP * CHUNK

    @functools.partial(
        pl.kernel,
        out_type=jax.ShapeDtypeStruct((ngrp, CHUNK), jnp.float32),
        mesh=plsc.VectorSubcoreMesh(**_MESH),
        scratch_types=[
            pltpu.VMEM((2, CHUNK * NF), jnp.float32),
            pltpu.VMEM((2, F_SP, CHUNK), jnp.int32),
            pltpu.VMEM((2, F_SP, CHUNK), jnp.int32),
            pltpu.VMEM((2 * nrow, D), jnp.float32),
            pltpu.VMEM((1, CHUNK), jnp.float32),
            pltpu.SemaphoreType.DMA,
            pltpu.SemaphoreType.DMA,
        ],
        compiler_params=_SC_PARAMS,
    )
    def k(in_hbm, s_hbm, out_hbm, raw_v, idx_v, off_v, rows_v, red_v, gsem, rsem):
        wid = lax.axis_index("c") * NS + lax.axis_index("s")
        lane = lax.iota(jnp.int32, 16)
        grp0 = wid * nchunk

        def fetch_raw(kc, slot):
            pltpu.async_copy(
                in_hbm.at[pl.ds((grp0 + kc) * CHUNK * NF, CHUNK * NF)],
                raw_v.at[slot],
                rsem,
            )

        def wait_raw(slot):
            pltpu.make_async_copy(
                in_hbm.at[pl.ds(0, CHUNK * NF)], raw_v.at[slot], rsem
            ).wait()

        def extract(slot):
            @pl.loop(0, F_SP)
            def _field(f):
                for g in range(NG):
                    pos = lane * NF + (g * 16 * NF) + f
                    ids = plsc.load_gather(raw_v.at[slot], [pos]).astype(jnp.int32)
                    p = ids + f * VPAD
                    idx_v[slot, f, pl.ds(g * 16, 16)] = p >> 4
                    off_v[slot, f, pl.ds(g * 16, 16)] = p & 15

        def fire_gathers(slot):
            @pl.loop(0, F_SP)
            def _field(f):
                pltpu.async_copy(
                    s_hbm.at[idx_v.at[slot, f]],
                    rows_v.at[pl.ds(slot * nrow + f * CHUNK, CHUNK), :],
                    gsem,
                )

        def drain_gathers():
            pltpu.make_async_copy(
                s_hbm.at[pl.ds(0, nrow), :],
                rows_v.at[pl.ds(0, nrow), :],
                gsem,
            ).wait()

        fetch_raw(0, 0)
        wait_raw(0)
        extract(0)
        fire_gathers(0)
        if nchunk > 1:
            fetch_raw(1, 1)

        @pl.loop(0, nchunk)
        def _chunk(kc):
            slot = lax.rem(kc, 2)

            @pl.when(kc + 1 < nchunk)
            def _():
                wait_raw(1 - slot)
                extract(1 - slot)

            drain_gathers()

            @pl.when(kc + 1 < nchunk)
            def _():
                fire_gathers(1 - slot)

            for g in range(NG):
                rbase = slot * nrow + lane + g * 16
                acc = plsc.load_gather(
                    rows_v, [rbase, off_v[slot, 0, pl.ds(g * 16, 16)]]
                )
                for f in range(1, F_SP):
                    acc = acc + plsc.load_gather(
                        rows_v,
                        [rbase + f * CHUNK, off_v[slot, f, pl.ds(g * 16, 16)]],
                    )
                red_v[0, pl.ds(g * 16, 16)] = acc

            @pl.when(kc + 2 < nchunk)
            def _():
                fetch_raw(kc + 2, slot)

            pltpu.sync_copy(red_v, out_hbm.at[pl.ds(grp0 + kc, 1), :])

    return k(inputs_flat, s16)


def _tc_combine(inputs, sp, gamma, beta, wt, bias):
    def body(in_ref, sp_ref, g_ref, b_ref, w_ref, bias_ref, out_ref):
        d = in_ref[:, F_SP:]
        mean = jnp.mean(d, axis=0, keepdims=True)
        c = d - mean
        var = jnp.mean(c * c, axis=0, keepdims=True)
        bn = c * lax.rsqrt(var + EPS) * g_ref[...][None, :] + b_ref[...][None, :]
        dense_logit = jnp.sum(bn * w_ref[...], axis=1, keepdims=True)
        out_ref[...] = sp_ref[...] + dense_logit + bias_ref[...][None, :]

    return pl.pallas_call(
        body,
        out_shape=jax.ShapeDtypeStruct((inputs.shape[0], 1), jnp.float32),
    )(inputs, sp, gamma, beta, wt, bias)


def kernel(inputs, tables, gamma, beta, W, bias):
    b = inputs.shape[0]
    s_flat = _tc_rowsum(jnp.transpose(tables, (0, 2, 1)))
    sp = _sc_fused(inputs.reshape(-1), s_flat.reshape(NROW16, D))
    wt = W.reshape(1, F_DN)
    return _tc_combine(inputs, sp.reshape(b, 1), gamma, beta, wt, bias)
